# baseline (speedup 1.0000x reference)
import functools

import jax
import jax.numpy as jnp
from jax.experimental import pallas as pl
from jax.experimental.pallas import tpu as pltpu

_NUM_CLASSES = 15
_TOPK = 10
_ALPHA = 0.5
_BETA = 6.0
_TEMPERATURE = 2.0
_EPS = 1e-9
_BIG = 3.0e38


def _body(d_ref, gt_ref, out_ref, *, A, G, C, GP):
    d = d_ref[0]
    ps = d[:, 0:C]
    gtr = gt_ref[0]
    gx = gtr[0:1, :]
    gy = gtr[1:2, :]
    gw = gtr[2:3, :]
    gh = gtr[3:4, :]
    gth = gtr[4:5, :]
    glab = gtr[5:6, :]
    gmsk = gtr[6:7, :]

    g_iota = jax.lax.broadcasted_iota(jnp.int32, (1, GP), 1).astype(jnp.float32)
    g_valid = g_iota < G
    a_iota = jax.lax.broadcasted_iota(jnp.int32, (A, 1), 0).astype(jnp.float32)
    row_iota = jax.lax.broadcasted_iota(
        jnp.int32, (16, GP), 0).astype(jnp.float32)

    px = d[:, C:C + 1]
    py = d[:, C + 1:C + 2]
    ax = d[:, C + 5:C + 6]
    ay = d[:, C + 6:C + 7]

    bs = jnp.clip(jax.nn.sigmoid(ps), _EPS, 1.0)
    c_iota = jax.lax.broadcasted_iota(jnp.int32, (C, GP), 0).astype(jnp.float32)
    onehot_cg = jnp.where(c_iota == glab, 1.0, 0.0)
    bscore = jnp.dot(bs, onehot_cg, preferred_element_type=jnp.float32,
                     precision=jax.lax.Precision.HIGHEST)

    scale = jnp.sqrt(gw * gh)
    inv_s1 = 1.0 / (scale + 1e-6)
    dx = px - gx
    dy = py - gy
    ov = jnp.exp(-((dx * dx + dy * dy) * (inv_s1 * inv_s1)))
    align = jnp.exp(_ALPHA * jnp.log(bscore + _EPS) +
                    _BETA * jnp.log(ov + _EPS))
    adx = ax - gx
    ady = ay - gy
    cd2 = adx * adx + ady * ady
    inv_s2 = 1.0 / (scale * 1.5 + 1e-6)
    cw = jnp.exp(-(cd2 * (inv_s2 * inv_s2)))
    am0 = align * cw

    cos_t = jnp.cos(-gth)
    sin_t = jnp.sin(-gth)
    ldx = adx * cos_t - ady * sin_t
    ldy = adx * sin_t + ady * cos_t
    iic = (jnp.abs(ldx) < gw * 1.5) & (jnp.abs(ldy) < gh * 1.5)
    iicf = jnp.where(iic, 1.0, 0.0)

    cpg = jnp.sum(iicf, axis=0, keepdims=True)
    area = gw * gh
    tiny_missed = (gmsk > 0) & (area < 0.01) & (cpg == 0.0) & g_valid
    fk = jnp.where(area < 0.002, 5.0, 3.0)

    def fb_body(r, carry):
        iicf, cdw = carry
        mn = jnp.min(cdw, axis=0, keepdims=True)
        fi = jnp.min(jnp.where(cdw == mn, a_iota, float(A)),
                     axis=0, keepdims=True)
        selr = a_iota == fi
        addg = tiny_missed & (r.astype(jnp.float32) < fk)
        iicf = jnp.where(selr & addg, 1.0, iicf)
        cdw = jnp.where(selr, _BIG, cdw)
        return iicf, cdw

    iicf, _ = jax.lax.fori_loop(0, 5, fb_body, (iicf, cd2))

    am = jnp.where(g_valid, am0 * iicf * gmsk, -1.0)

    rm = jnp.max(am, axis=1, keepdims=True)
    fg = jnp.min(jnp.where(am == rm, g_iota, float(GP)),
                 axis=1, keepdims=True)
    selg = g_iota == fg

    def gat(row):
        return jnp.sum(jnp.where(selg, row, 0.0), axis=1, keepdims=True)

    out_ref[0, :, 21:22] = gat(glab)
    for j in range(5):
        out_ref[0, :, 16 + j:17 + j] = gat(gtr[j:j + 1, :])
    ov_sel = jnp.sum(jnp.where(selg, ov, 0.0), axis=1, keepdims=True)

    def tk_body(r, carry):
        work, ovsum, am_max, rnk = carry
        m = jnp.max(work, axis=0, keepdims=True)
        fi = jnp.min(jnp.where(work == m, a_iota, float(A)),
                     axis=0, keepdims=True)
        selr = a_iota == fi
        ovsum = ovsum + jnp.sum(jnp.where(selr, ov, 0.0), axis=0,
                                keepdims=True)
        rf = r.astype(jnp.float32)
        am_max = jnp.where(rf == 0.0, m, am_max)
        rnk = jnp.where(selr, rf, rnk)
        work = jnp.where(selr, -2.0, work)
        return work, ovsum, am_max, rnk

    _, ovsum, am_max, rnk = jax.lax.fori_loop(
        0, _TOPK, tk_body,
        (am, jnp.zeros((1, GP), jnp.float32),
         jnp.zeros((1, GP), jnp.float32),
         jnp.full((A, GP), 99.0, jnp.float32)))
    dk = jnp.clip(jnp.round(ovsum), 1.0, float(_TOPK))

    pos = (rnk < dk) & (am > _EPS) & g_valid
    ipm = jnp.max(jnp.where(pos, 1.0, 0.0), axis=1, keepdims=True)

    selg2 = g_iota == fg
    amax_sel = jnp.sum(jnp.where(selg2, am_max, 0.0), axis=1, keepdims=True)
    raw = rm / (amax_sel + _EPS) * ov_sel

    tlab = out_ref[0, :, 21:22]
    valid = (tlab >= 0) & (tlab < C) & (ipm > 0)
    sval = jnp.where(valid, jnp.sqrt(raw), 0.0)
    c_iota_r = jax.lax.broadcasted_iota(jnp.int32, (1, C), 1).astype(jnp.float32)
    out_ref[0, :, 0:C] = jnp.where(c_iota_r == tlab, sval, 0.0)
    out_ref[0, :, 22:23] = ipm


def kernel(pred_scores, pred_bboxes, anchor_points, gt_labels, gt_bboxes,
           mask_gt):
    B, A, C = pred_scores.shape
    G = gt_bboxes.shape[1]
    GP = ((G + 127) // 128) * 128
    gt_rows = jnp.concatenate([
        jnp.transpose(gt_bboxes, (0, 2, 1)),
        gt_labels.reshape(B, G)[:, None, :].astype(jnp.float32),
        mask_gt.reshape(B, G)[:, None, :].astype(jnp.float32),
        jnp.zeros((B, 1, G), jnp.float32),
    ], axis=1)
    gt_rows = jnp.pad(gt_rows, ((0, 0), (0, 0), (0, GP - G)))
    dense = jnp.concatenate([
        pred_scores, pred_bboxes,
        jnp.broadcast_to(anchor_points[None], (B, A, 2)),
    ], axis=2)

    out = pl.pallas_call(
        functools.partial(_body, A=A, G=G, C=C, GP=GP),
        grid=(B,),
        in_specs=[
            pl.BlockSpec((1, A, C + 7), lambda b: (b, 0, 0)),
            pl.BlockSpec((1, 8, GP), lambda b: (b, 0, 0)),
        ],
        out_specs=pl.BlockSpec((1, A, 32), lambda b: (b, 0, 0)),
        out_shape=jax.ShapeDtypeStruct((B, A, 32), jnp.float32),
        compiler_params=pltpu.CompilerParams(
            vmem_limit_bytes=100 * 1024 * 1024),
    )(dense, gt_rows)
    tscore = out[:, :, 0:C]
    tbb = out[:, :, C + 1:C + 6]
    tlab = out[:, :, C + 6].astype(jnp.int32)
    ipm = out[:, :, C + 7] > 0
    return tlab, tbb, tscore, ipm


# speedup vs baseline: 1.0609x; 1.0609x over previous
import functools

import jax
import jax.numpy as jnp
from jax.experimental import pallas as pl
from jax.experimental.pallas import tpu as pltpu

_NUM_CLASSES = 15
_TOPK = 10
_ALPHA = 0.5
_BETA = 6.0
_TEMPERATURE = 2.0
_EPS = 1e-9
_BIG = 3.0e38


def _body(d_ref, gt_ref, out_ref, *, A, G, C, GP):
    d = d_ref[0]
    ps = d[:, 0:C]
    gtr = gt_ref[0]
    gx = gtr[0:1, :]
    gy = gtr[1:2, :]
    gw = gtr[2:3, :]
    gh = gtr[3:4, :]
    gth = gtr[4:5, :]
    glab = gtr[5:6, :]
    gmsk = gtr[6:7, :]

    g_iota = jax.lax.broadcasted_iota(jnp.int32, (1, GP), 1).astype(jnp.float32)
    g_valid = g_iota < G
    a_iota = jax.lax.broadcasted_iota(jnp.int32, (A, 1), 0).astype(jnp.float32)
    row_iota = jax.lax.broadcasted_iota(
        jnp.int32, (16, GP), 0).astype(jnp.float32)

    px = d[:, C:C + 1]
    py = d[:, C + 1:C + 2]
    ax = d[:, C + 5:C + 6]
    ay = d[:, C + 6:C + 7]

    bs = jnp.clip(jax.nn.sigmoid(ps), _EPS, 1.0)
    c_iota = jax.lax.broadcasted_iota(jnp.int32, (C, GP), 0).astype(jnp.float32)
    onehot_cg = jnp.where(c_iota == glab, 1.0, 0.0)
    bscore = jnp.dot(bs, onehot_cg, preferred_element_type=jnp.float32,
                     precision=jax.lax.Precision.HIGHEST)

    scale = jnp.sqrt(gw * gh)
    inv_s1 = 1.0 / (scale + 1e-6)
    dx = px - gx
    dy = py - gy
    ov = jnp.exp(-((dx * dx + dy * dy) * (inv_s1 * inv_s1)))
    align = jnp.exp(_ALPHA * jnp.log(bscore + _EPS) +
                    _BETA * jnp.log(ov + _EPS))
    adx = ax - gx
    ady = ay - gy
    cd2 = adx * adx + ady * ady
    inv_s2 = 1.0 / (scale * 1.5 + 1e-6)
    cw = jnp.exp(-(cd2 * (inv_s2 * inv_s2)))
    am0 = align * cw

    cos_t = jnp.cos(-gth)
    sin_t = jnp.sin(-gth)
    ldx = adx * cos_t - ady * sin_t
    ldy = adx * sin_t + ady * cos_t
    iic = (jnp.abs(ldx) < gw * 1.5) & (jnp.abs(ldy) < gh * 1.5)
    iicf = jnp.where(iic, 1.0, 0.0)

    cpg = jnp.sum(iicf, axis=0, keepdims=True)
    area = gw * gh
    tiny_missed = (gmsk > 0) & (area < 0.01) & (cpg == 0.0) & g_valid
    fk = jnp.where(area < 0.002, 5.0, 3.0)

    def fb_body(r, carry):
        iicf, cdw = carry
        mn = jnp.min(cdw, axis=0, keepdims=True)
        fi = jnp.min(jnp.where(cdw == mn, a_iota, float(A)),
                     axis=0, keepdims=True)
        selr = a_iota == fi
        addg = tiny_missed & (r.astype(jnp.float32) < fk)
        iicf = jnp.where(selr & addg, 1.0, iicf)
        cdw = jnp.where(selr, _BIG, cdw)
        return iicf, cdw

    iicf, _ = jax.lax.fori_loop(0, 5, fb_body, (iicf, cd2))

    am = jnp.where(g_valid, am0 * iicf * gmsk, -1.0)

    rm = jnp.max(am, axis=1, keepdims=True)
    fg = jnp.min(jnp.where(am == rm, g_iota, float(GP)),
                 axis=1, keepdims=True)
    selg = g_iota == fg

    def gat(row):
        return jnp.sum(jnp.where(selg, row, 0.0), axis=1, keepdims=True)

    out_ref[0, :, 21:22] = gat(glab)
    for j in range(5):
        out_ref[0, :, 16 + j:17 + j] = gat(gtr[j:j + 1, :])
    ov_sel = jnp.sum(jnp.where(selg, ov, 0.0), axis=1, keepdims=True)

    def tk_body(r, carry):
        work, ovsum, vals, idxs = carry
        m = jnp.max(work, axis=0, keepdims=True)
        fi = jnp.min(jnp.where(work == m, a_iota, float(A)),
                     axis=0, keepdims=True)
        selr = a_iota == fi
        ovsum = ovsum + jnp.sum(jnp.where(selr, ov, 0.0), axis=0,
                                keepdims=True)
        upd = row_iota == r.astype(jnp.float32)
        vals = jnp.where(upd, m, vals)
        idxs = jnp.where(upd, fi, idxs)
        work = jnp.where(selr, -2.0, work)
        return work, ovsum, vals, idxs

    _, ovsum, vals, idxs = jax.lax.fori_loop(
        0, _TOPK, tk_body,
        (am, jnp.zeros((1, GP), jnp.float32),
         jnp.full((16, GP), -3.0, jnp.float32),
         jnp.full((16, GP), float(A), jnp.float32)))
    am_max = vals[0:1, :]
    dk = jnp.clip(jnp.round(ovsum), 1.0, float(_TOPK))

    def ip_body(r, ipm):
        rf = r.astype(jnp.float32)
        sel_row = row_iota == rf
        vr = jnp.sum(jnp.where(sel_row, vals, 0.0), axis=0, keepdims=True)
        ir = jnp.sum(jnp.where(sel_row, idxs, 0.0), axis=0, keepdims=True)
        qual = (vr > _EPS) & (rf < dk) & g_valid
        hit = (a_iota == ir) & qual
        return jnp.maximum(
            ipm, jnp.max(jnp.where(hit, 1.0, 0.0), axis=1, keepdims=True))

    ipm = jax.lax.fori_loop(0, _TOPK, ip_body,
                            jnp.zeros((A, 1), jnp.float32))

    selg2 = g_iota == fg
    amax_sel = jnp.sum(jnp.where(selg2, am_max, 0.0), axis=1, keepdims=True)
    raw = rm / (amax_sel + _EPS) * ov_sel

    tlab = out_ref[0, :, 21:22]
    valid = (tlab >= 0) & (tlab < C) & (ipm > 0)
    sval = jnp.where(valid, jnp.sqrt(raw), 0.0)
    c_iota_r = jax.lax.broadcasted_iota(jnp.int32, (1, C), 1).astype(jnp.float32)
    out_ref[0, :, 0:C] = jnp.where(c_iota_r == tlab, sval, 0.0)
    out_ref[0, :, 22:23] = ipm


def kernel(pred_scores, pred_bboxes, anchor_points, gt_labels, gt_bboxes,
           mask_gt):
    B, A, C = pred_scores.shape
    G = gt_bboxes.shape[1]
    GP = ((G + 127) // 128) * 128
    gt_rows = jnp.concatenate([
        jnp.transpose(gt_bboxes, (0, 2, 1)),
        gt_labels.reshape(B, G)[:, None, :].astype(jnp.float32),
        mask_gt.reshape(B, G)[:, None, :].astype(jnp.float32),
        jnp.zeros((B, 1, G), jnp.float32),
    ], axis=1)
    gt_rows = jnp.pad(gt_rows, ((0, 0), (0, 0), (0, GP - G)))
    dense = jnp.concatenate([
        pred_scores, pred_bboxes,
        jnp.broadcast_to(anchor_points[None], (B, A, 2)),
    ], axis=2)

    out = pl.pallas_call(
        functools.partial(_body, A=A, G=G, C=C, GP=GP),
        grid=(B,),
        in_specs=[
            pl.BlockSpec((1, A, C + 7), lambda b: (b, 0, 0)),
            pl.BlockSpec((1, 8, GP), lambda b: (b, 0, 0)),
        ],
        out_specs=pl.BlockSpec((1, A, 32), lambda b: (b, 0, 0)),
        out_shape=jax.ShapeDtypeStruct((B, A, 32), jnp.float32),
        compiler_params=pltpu.CompilerParams(
            vmem_limit_bytes=100 * 1024 * 1024),
    )(dense, gt_rows)
    tscore = out[:, :, 0:C]
    tbb = out[:, :, C + 1:C + 6]
    tlab = out[:, :, C + 6].astype(jnp.int32)
    ipm = out[:, :, C + 7] > 0
    return tlab, tbb, tscore, ipm
